# Initial kernel scaffold; baseline (speedup 1.0000x reference)
import jax
import jax.numpy as jnp
from jax.experimental import pallas as pl


def kernel(x, ptr):
    raise NotImplementedError("write your pallas kernel here")


# speedup vs baseline: 5.1486x; 5.1486x over previous
import functools

import jax
import jax.numpy as jnp
from jax import lax
from jax.experimental import pallas as pl
from jax.experimental.pallas import tpu as pltpu
from jax.experimental.pallas import tpu_sc as plsc

N_ROWS = 32768
D = 128
NSEG = 16
PAD_COLS = 34
LANES = 16
NGROUP = D // LANES

NC = 2
NS = 16
NW = NC * NS
ROWS_PER_W = N_ROWS // NW
CHUNK = 256
NCHUNK = ROWS_PER_W // CHUNK


def _sc_partial_sums(x, ptr):
  mesh = plsc.VectorSubcoreMesh(core_axis_name="c", subcore_axis_name="s")

  @functools.partial(
      pl.kernel,
      mesh=mesh,
      out_type=jax.ShapeDtypeStruct((NW, NSEG, D), jnp.float32),
      scratch_types=[
          pltpu.VMEM((CHUNK, D), jnp.float32),
          pltpu.VMEM((CHUNK, D), jnp.float32),
          pltpu.VMEM((LANES,), jnp.int32),
          pltpu.VMEM((NSEG, D), jnp.float32),
          pltpu.SemaphoreType.DMA,
          pltpu.SemaphoreType.DMA,
      ],
  )
  def k(x_hbm, ptr_hbm, out_hbm, buf0, buf1, ptr_v, acc, sem0, sem1):
    wid = lax.axis_index("s") * NC + lax.axis_index("c")
    lo = wid * ROWS_PER_W

    pltpu.sync_copy(ptr_hbm.at[pl.ds(0, LANES)], ptr_v)
    ptrv = ptr_v[...]
    ps = [ptrv[s] for s in range(NSEG)]
    ps.append(jnp.int32(N_ROWS))

    zeros = jnp.zeros((LANES,), jnp.float32)
    for s in range(NSEG):
      for g in range(NGROUP):
        acc[s, pl.ds(g * LANES, LANES)] = zeros

    bufs = (buf0, buf1)
    sems = (sem0, sem1)

    def chunk_copy(c, buf, sem):
      return pltpu.make_async_copy(
          x_hbm.at[pl.ds(lo + c * CHUNK, CHUNK), :], buf, sem)

    chunk_copy(0, bufs[0], sems[0]).start()

    for c in range(NCHUNK):
      buf = bufs[c % 2]
      chunk_copy(c, buf, sems[c % 2]).wait()
      if c + 1 < NCHUNK:
        chunk_copy(c + 1, bufs[(c + 1) % 2], sems[(c + 1) % 2]).start()

      clo = lo + c * CHUNK
      chi = clo + CHUNK
      for s in range(NSEG):
        a = jnp.maximum(ps[s], clo)
        b = jnp.minimum(ps[s + 1], chi)

        def body(i, carry):
          j = i - clo
          return tuple(
              carry[g] + buf[j, pl.ds(g * LANES, LANES)]
              for g in range(NGROUP))

        init = tuple(zeros for _ in range(NGROUP))
        part = lax.fori_loop(a, b, body, init)
        for g in range(NGROUP):
          sl = pl.ds(g * LANES, LANES)
          acc[s, sl] = acc[s, sl] + part[g]

    pltpu.sync_copy(acc, out_hbm.at[wid])

  return k(x, ptr)


def _combine_kernel(p_ref, cnt_ref, out_ref):
  sums = jnp.sum(p_ref[...], axis=0)
  mean = sums / cnt_ref[...]
  out_ref[...] = jnp.concatenate(
      [mean, jnp.zeros((NSEG, PAD_COLS), jnp.float32)], axis=1)


def kernel(x, ptr):
  partials = _sc_partial_sums(x, ptr)
  cnt = jnp.maximum(ptr[1:] - ptr[:-1], 1).astype(jnp.float32).reshape(NSEG, 1)
  out = pl.pallas_call(
      _combine_kernel,
      out_shape=jax.ShapeDtypeStruct((NSEG, D + PAD_COLS), jnp.float32),
  )(partials, cnt)
  return out


# speedup vs baseline: 6.5947x; 1.2809x over previous
import functools

import jax
import jax.numpy as jnp
from jax import lax
from jax.experimental import pallas as pl
from jax.experimental.pallas import tpu as pltpu
from jax.experimental.pallas import tpu_sc as plsc

N_ROWS = 32768
D = 128
NSEG = 16
PAD_COLS = 34
LANES = 16
NGROUP = D // LANES

NC = 2
NS = 16
NW = NC * NS
ROWS_PER_W = N_ROWS // NW
CHUNK = 256
NCHUNK = ROWS_PER_W // CHUNK


def _sc_partial_sums(x, ptr):
  mesh = plsc.VectorSubcoreMesh(core_axis_name="c", subcore_axis_name="s")

  @functools.partial(
      pl.kernel,
      mesh=mesh,
      out_type=jax.ShapeDtypeStruct((NW, NSEG, D), jnp.float32),
      scratch_types=[
          pltpu.VMEM((CHUNK, D), jnp.float32),
          pltpu.VMEM((CHUNK, D), jnp.float32),
          pltpu.VMEM((LANES,), jnp.int32),
          pltpu.VMEM((NSEG, D), jnp.float32),
          pltpu.SMEM((NSEG + 1,), jnp.int32),
          pltpu.SemaphoreType.DMA,
          pltpu.SemaphoreType.DMA,
      ],
  )
  def k(x_hbm, ptr_hbm, out_hbm, buf0, buf1, ptr_v, acc, psm, sem0, sem1):
    wid = lax.axis_index("s") * NC + lax.axis_index("c")
    lo = wid * ROWS_PER_W

    pltpu.sync_copy(ptr_hbm.at[pl.ds(0, LANES)], ptr_v)
    ptrv = ptr_v[...]
    for s in range(NSEG):
      psm[s] = ptrv[s]
    psm[NSEG] = jnp.int32(N_ROWS)

    zeros = jnp.zeros((LANES,), jnp.float32)

    def zero_body(s, _):
      for g in range(NGROUP):
        acc[s, pl.ds(g * LANES, LANES)] = zeros
      return 0

    lax.fori_loop(0, NSEG, zero_body, 0)

    def chunk_copy(c, buf, sem):
      return pltpu.make_async_copy(
          x_hbm.at[pl.ds(lo + c * CHUNK, CHUNK), :], buf, sem)

    def process(buf, c):
      clo = lo + c * CHUNK

      def seg_body(s, _):
        a = jnp.maximum(psm[s], clo)
        b = jnp.minimum(psm[s + 1], clo + CHUNK)

        @pl.when(a < b)
        def _():
          def row_body(i, carry):
            j = i - clo
            return tuple(
                carry[g] + buf[j, pl.ds(g * LANES, LANES)]
                for g in range(NGROUP))

          part = lax.fori_loop(a, b, row_body,
                               tuple(zeros for _ in range(NGROUP)))
          for g in range(NGROUP):
            sl = pl.ds(g * LANES, LANES)
            acc[s, sl] = acc[s, sl] + part[g]

        return 0

      lax.fori_loop(0, NSEG, seg_body, 0)

    chunk_copy(0, buf0, sem0).start()
    chunk_copy(1, buf1, sem1).start()

    def pair_body(i, _):
      c = 2 * i
      chunk_copy(c, buf0, sem0).wait()
      process(buf0, c)

      @pl.when(c + 2 < NCHUNK)
      def _():
        chunk_copy(c + 2, buf0, sem0).start()

      chunk_copy(c + 1, buf1, sem1).wait()
      process(buf1, c + 1)

      @pl.when(c + 3 < NCHUNK)
      def _():
        chunk_copy(c + 3, buf1, sem1).start()

      return 0

    lax.fori_loop(0, NCHUNK // 2, pair_body, 0)

    pltpu.sync_copy(acc, out_hbm.at[wid])

  return k(x, ptr)


def _combine_kernel(p_ref, cnt_ref, out_ref):
  sums = jnp.sum(p_ref[...], axis=0)
  mean = sums / cnt_ref[...]
  out_ref[...] = jnp.concatenate(
      [mean, jnp.zeros((NSEG, PAD_COLS), jnp.float32)], axis=1)


def kernel(x, ptr):
  partials = _sc_partial_sums(x, ptr)
  cnt = jnp.maximum(ptr[1:] - ptr[:-1], 1).astype(jnp.float32).reshape(NSEG, 1)
  out = pl.pallas_call(
      _combine_kernel,
      out_shape=jax.ShapeDtypeStruct((NSEG, D + PAD_COLS), jnp.float32),
  )(partials, cnt)
  return out
